# baseline (speedup 1.0000x reference)
import functools

import jax
import jax.numpy as jnp
from jax import lax
from jax.experimental import pallas as pl
from jax.experimental.pallas import tpu as pltpu
from jax.experimental.pallas import tpu_sc as plsc

N = 10000
E = 160000
D = 364
OUT = 150
B = 4096

N_PAD = 10240
D_PAD = 384
SL = 128
E_PAD = 163840
EPT = E_PAD // 16
ECHUNK = 128
SEGC = 40
ESEG = SEGC * ECHUNK
RPT = N_PAD // 16
OUT_PAD = 256
BB = 2 * B
VPT = BB // 16

_SC_MESH = dict(core_axis_name="c", subcore_axis_name="s")


def _leaky(x):
    return jnp.where(x > 0, x, 0.01 * x)


def _sc_segsum_body(x0, x1, x2, src_hbm, dst_hbm,
                    o0, o1, o2a, o2b,
                    src_all, dst_all, didx, gbuf, acc, sem):
    c = lax.axis_index("c")
    s = lax.axis_index("s")
    rbase = s * RPT

    def run_segment(table, ebase):
        pltpu.sync_copy(src_hbm.at[pl.ds(ebase, ESEG)], src_all)
        pltpu.sync_copy(dst_hbm.at[pl.ds(ebase, ESEG)], dst_all)

        def fill(i, b):
            for j in range(8):
                off = i * ECHUNK + j * 16
                didx[b, pl.ds(j * 16, 16)] = dst_all[pl.ds(off, 16)]

        def gather(i, b):
            return pltpu.async_copy(
                table.at[src_all.at[pl.ds(i * ECHUNK, ECHUNK)]],
                gbuf.at[b], sem.at[b])

        def gather_wait(i, b):
            pltpu.make_async_copy(
                table.at[src_all.at[pl.ds(i * ECHUNK, ECHUNK)]],
                gbuf.at[b], sem.at[b]).wait()

        def scatter_add(b):
            pltpu.sync_copy(gbuf.at[b], acc.at[didx.at[b]], add=True)

        fill(0, 0)
        gather(0, 0)

        def pair(k, _):
            i0 = 2 * k
            fill(i0 + 1, 1)
            gather_wait(i0, 0)
            gather(i0 + 1, 1)
            scatter_add(0)

            @pl.when(k < SEGC // 2 - 1)
            def _():
                fill(i0 + 2, 0)
                gather(i0 + 2, 0)
            gather_wait(i0 + 1, 1)
            scatter_add(1)
            return 0

        lax.fori_loop(0, SEGC // 2, pair, 0)

    def run_slice(table, out, ebase, nseg):
        for i in range(RPT // ECHUNK):
            b = i % 2
            r0 = rbase + i * ECHUNK
            pltpu.sync_copy(table.at[pl.ds(r0, ECHUNK)], gbuf.at[b])
            pltpu.sync_copy(gbuf.at[b], acc.at[pl.ds(r0, ECHUNK)])
        plsc.subcore_barrier()

        for seg in range(nseg):
            run_segment(table, ebase + seg * ESEG)
        plsc.subcore_barrier()

        for i in range(RPT // ECHUNK):
            b = i % 2
            r0 = rbase + i * ECHUNK
            pltpu.sync_copy(acc.at[pl.ds(r0, ECHUNK)], gbuf.at[b])
            pltpu.sync_copy(gbuf.at[b], out.at[pl.ds(r0, ECHUNK)])
        plsc.subcore_barrier()

    @pl.when(c == 0)
    def _():
        run_slice(x0, o0, s * EPT, 2)
        run_slice(x2, o2a, s * (EPT // 2), 1)

    @pl.when(c == 1)
    def _():
        run_slice(x1, o1, s * EPT, 2)
        run_slice(x2, o2b, E_PAD // 2 + s * (EPT // 2), 1)


@functools.cache
def _sc_segsum():
    st = jax.ShapeDtypeStruct((N_PAD, SL), jnp.float32)
    return pl.kernel(
        _sc_segsum_body,
        mesh=plsc.VectorSubcoreMesh(**_SC_MESH),
        compiler_params=pltpu.CompilerParams(use_tc_tiling_on_sc=False),
        out_type=(st, st, st, st),
        scratch_types=[
            pltpu.VMEM((ESEG,), jnp.int32),
            pltpu.VMEM((ESEG,), jnp.int32),
            pltpu.VMEM((2, ECHUNK), jnp.int32),
            pltpu.VMEM((2, ECHUNK, SL), jnp.float32),
            pltpu.VMEM_SHARED((N_PAD, SL), jnp.float32),
            pltpu.SemaphoreType.DMA((2,)),
        ],
    )


def _sc_gather_body(t0, t1, t2a, t2b, h2, vcat_hbm,
                    u0, u1, u2a, u2b, uh2,
                    vcat_v, gbuf, sem):
    c = lax.axis_index("c")
    s = lax.axis_index("s")
    vbase = s * VPT
    pltpu.sync_copy(vcat_hbm.at[pl.ds(vbase, VPT)], vcat_v)

    def run(table, out):
        for k in range(VPT // ECHUNK):
            pltpu.async_copy(
                table.at[vcat_v.at[pl.ds(k * ECHUNK, ECHUNK)]],
                gbuf, sem).wait()
            pltpu.sync_copy(
                gbuf, out.at[pl.ds(vbase + k * ECHUNK, ECHUNK)])

    @pl.when(c == 0)
    def _():
        run(t0, u0)
        run(t2a, u2a)
        run(h2, uh2)

    @pl.when(c == 1)
    def _():
        run(t1, u1)
        run(t2b, u2b)


@functools.cache
def _sc_gather():
    st = jax.ShapeDtypeStruct((BB, SL), jnp.float32)
    return pl.kernel(
        _sc_gather_body,
        mesh=plsc.VectorSubcoreMesh(**_SC_MESH),
        compiler_params=pltpu.CompilerParams(use_tc_tiling_on_sc=False),
        out_type=(st, st, st, st, st),
        scratch_types=[
            pltpu.VMEM((VPT,), jnp.int32),
            pltpu.VMEM((ECHUNK, SL), jnp.float32),
            pltpu.SemaphoreType.DMA,
        ],
    )


def _tc_linear_body(p0, p1, p2a, p2b, x2_ref, w_ref, b_ref, o0, o1, o2):
    sblk = jnp.concatenate(
        [p0[...], p1[...], p2a[...] + p2b[...] - x2_ref[...]], axis=1)
    acc = jnp.dot(sblk, w_ref[...], preferred_element_type=jnp.float32)
    h = _leaky(acc + b_ref[...])
    o0[...] = h[:, 0 * SL:1 * SL]
    o1[...] = h[:, 1 * SL:2 * SL]
    o2[...] = h[:, 2 * SL:3 * SL]


def _tc_linear(p0, p1, p2a, p2b, x2, wt, b):
    n = p0.shape[0]
    blk = 512
    st = jax.ShapeDtypeStruct((n, SL), jnp.float32)
    bs = pl.BlockSpec((blk, SL), lambda i: (i, 0))
    return pl.pallas_call(
        _tc_linear_body,
        grid=(n // blk,),
        in_specs=[bs, bs, bs, bs, bs,
                  pl.BlockSpec((D_PAD, D_PAD), lambda i: (0, 0)),
                  pl.BlockSpec((1, D_PAD), lambda i: (0, 0))],
        out_specs=[bs, bs, bs],
        out_shape=[st, st, st],
    )(p0, p1, p2a, p2b, x2, wt, b)


def _tc_final_body(u0, u1, u2a, u2b, uh2, w2_ref, b2_ref, w3_ref, b3_ref,
                   o_ref):
    u = jnp.concatenate(
        [u0[...], u1[...], u2a[...] + u2b[...] - uh2[...]], axis=1)
    t = jnp.dot(u, w2_ref[...], preferred_element_type=jnp.float32)
    t = _leaky(t + b2_ref[...])
    z = jnp.dot(t, w3_ref[...], preferred_element_type=jnp.float32)
    z = _leaky(z + b3_ref[...])
    nrm = jnp.sqrt(jnp.sum(z * z, axis=1, keepdims=True))
    o_ref[...] = z / jnp.maximum(nrm, 1e-12)


def _tc_final(u0, u1, u2a, u2b, uh2, w2t, b2, w3t, b3):
    blk = 512
    bs = pl.BlockSpec((blk, SL), lambda i: (i, 0))
    return pl.pallas_call(
        _tc_final_body,
        grid=(BB // blk,),
        in_specs=[bs, bs, bs, bs, bs,
                  pl.BlockSpec((D_PAD, D_PAD), lambda i: (0, 0)),
                  pl.BlockSpec((1, D_PAD), lambda i: (0, 0)),
                  pl.BlockSpec((D_PAD, OUT_PAD), lambda i: (0, 0)),
                  pl.BlockSpec((1, OUT_PAD), lambda i: (0, 0))],
        out_specs=pl.BlockSpec((blk, OUT_PAD), lambda i: (i, 0)),
        out_shape=jax.ShapeDtypeStruct((BB, OUT_PAD), jnp.float32),
    )(u0, u1, u2a, u2b, uh2, w2t, b2, w3t, b3)


def kernel(features, edge_index, v1, v2, W1, b1, W2, b2, W3, b3):
    xp = jnp.pad(features, ((0, N_PAD - N), (0, D_PAD - D)))
    x0, x1, x2 = xp[:, :SL], xp[:, SL:2 * SL], xp[:, 2 * SL:]
    src = jnp.pad(edge_index[0], (0, E_PAD - E))
    dst = jnp.pad(edge_index[1], (0, E_PAD - E), constant_values=N_PAD - 1)
    vcat = jnp.concatenate([v1, v2])

    w1t = jnp.pad(W1, ((0, D_PAD - D), (0, D_PAD - D))).T
    b1p = jnp.pad(b1, (0, D_PAD - D)).reshape(1, D_PAD)
    w2t = jnp.pad(W2, ((0, D_PAD - D), (0, D_PAD - D))).T
    b2p = jnp.pad(b2, (0, D_PAD - D)).reshape(1, D_PAD)
    w3t = jnp.pad(W3, ((0, OUT_PAD - OUT), (0, D_PAD - D))).T
    b3p = jnp.pad(b3, (0, OUT_PAD - OUT)).reshape(1, OUT_PAD)

    s0, s1, s2a, s2b = _sc_segsum()(x0, x1, x2, src, dst)
    h0, h1, h2 = _tc_linear(s0, s1, s2a, s2b, x2, w1t, b1p)

    t0, t1, t2a, t2b = _sc_segsum()(h0, h1, h2, src, dst)

    u0, u1, u2a, u2b, uh2 = _sc_gather()(t0, t1, t2a, t2b, h2, vcat)
    z = _tc_final(u0, u1, u2a, u2b, uh2, w2t, b2p, w3t, b3p)
    return (z[:B, :OUT], z[B:, :OUT])


# speedup vs baseline: 1.0059x; 1.0059x over previous
import functools

import jax
import jax.numpy as jnp
from jax import lax
from jax.experimental import pallas as pl
from jax.experimental.pallas import tpu as pltpu
from jax.experimental.pallas import tpu_sc as plsc

N = 10000
E = 160000
D = 364
OUT = 150
B = 4096

N_PAD = 10240
D_PAD = 384
SL = 128
E_PAD = 163840
EPT = E_PAD // 16
ECHUNK = 64
SEGC = 80
ESEG = SEGC * ECHUNK
NRING = 4
RPT = N_PAD // 16
OUT_PAD = 256
BB = 2 * B
VPT = BB // 16

_SC_MESH = dict(core_axis_name="c", subcore_axis_name="s")


def _leaky(x):
    return jnp.where(x > 0, x, 0.01 * x)


def _sc_segsum_body(x0, x1, x2, src_hbm, dst_hbm,
                    o0, o1, o2a, o2b,
                    src_all, dst_all, didx, gbuf, acc, gsem, ssem):
    c = lax.axis_index("c")
    s = lax.axis_index("s")
    rbase = s * RPT

    def run_segment(table, ebase):
        pltpu.sync_copy(src_hbm.at[pl.ds(ebase, ESEG)], src_all)
        pltpu.sync_copy(dst_hbm.at[pl.ds(ebase, ESEG)], dst_all)

        def fill(i, t):
            for j in range(ECHUNK // 16):
                didx[t, pl.ds(j * 16, 16)] = dst_all[
                    pl.ds(i * ECHUNK + j * 16, 16)]

        def gather(i, t):
            pltpu.async_copy(
                table.at[src_all.at[pl.ds(i * ECHUNK, ECHUNK)]],
                gbuf.at[t], gsem.at[t])

        def gather_wait(i, t):
            pltpu.make_async_copy(
                table.at[src_all.at[pl.ds(i * ECHUNK, ECHUNK)]],
                gbuf.at[t], gsem.at[t]).wait()

        def scatter(t):
            pltpu.async_copy(gbuf.at[t], acc.at[didx.at[t]], ssem.at[t],
                             add=True)

        def scatter_wait(t):
            pltpu.make_async_copy(gbuf.at[t], acc.at[didx.at[t]],
                                  ssem.at[t]).wait()

        fill(0, 0)
        gather(0, 0)

        def group(g, _):
            i0 = NRING * g
            for t in range(NRING):
                i = i0 + t
                nxt = (t + 1) % NRING

                @pl.when(i + 1 < SEGC)
                def _():
                    @pl.when(i + 1 >= NRING)
                    def _():
                        scatter_wait(nxt)
                    fill(i + 1, nxt)
                    gather(i + 1, nxt)
                gather_wait(i, t)
                scatter(t)
            return 0

        lax.fori_loop(0, SEGC // NRING, group, 0)
        for t in range(NRING):
            scatter_wait(t)

    def pipe_copy(src, dst):
        nblk = RPT // ECHUNK

        def blk(ref, i):
            return ref.at[pl.ds(rbase + i * ECHUNK, ECHUNK)]

        for i in range(nblk):
            t = i % NRING
            if i >= NRING:
                pltpu.make_async_copy(gbuf.at[t], blk(dst, i - NRING),
                                      ssem.at[t]).wait()
            pltpu.async_copy(blk(src, i), gbuf.at[t], gsem.at[t])
            if i >= 1:
                tp = (i - 1) % NRING
                pltpu.make_async_copy(blk(src, i - 1), gbuf.at[tp],
                                      gsem.at[tp]).wait()
                pltpu.async_copy(gbuf.at[tp], blk(dst, i - 1), ssem.at[tp])
        tl = (nblk - 1) % NRING
        pltpu.make_async_copy(blk(src, nblk - 1), gbuf.at[tl],
                              gsem.at[tl]).wait()
        pltpu.async_copy(gbuf.at[tl], blk(dst, nblk - 1), ssem.at[tl])
        for i in range(max(0, nblk - NRING), nblk):
            t = i % NRING
            pltpu.make_async_copy(gbuf.at[t], blk(dst, i), ssem.at[t]).wait()

    def run_slice(table, out, ebase, nseg):
        pipe_copy(table, acc)
        plsc.subcore_barrier()

        for seg in range(nseg):
            run_segment(table, ebase + seg * ESEG)
        plsc.subcore_barrier()

        pipe_copy(acc, out)
        plsc.subcore_barrier()

    @pl.when(c == 0)
    def _():
        run_slice(x0, o0, s * EPT, 2)
        run_slice(x2, o2a, s * (EPT // 2), 1)

    @pl.when(c == 1)
    def _():
        run_slice(x1, o1, s * EPT, 2)
        run_slice(x2, o2b, E_PAD // 2 + s * (EPT // 2), 1)


@functools.cache
def _sc_segsum():
    st = jax.ShapeDtypeStruct((N_PAD, SL), jnp.float32)
    return pl.kernel(
        _sc_segsum_body,
        mesh=plsc.VectorSubcoreMesh(**_SC_MESH),
        compiler_params=pltpu.CompilerParams(use_tc_tiling_on_sc=False),
        out_type=(st, st, st, st),
        scratch_types=[
            pltpu.VMEM((ESEG,), jnp.int32),
            pltpu.VMEM((ESEG,), jnp.int32),
            pltpu.VMEM((NRING, ECHUNK), jnp.int32),
            pltpu.VMEM((NRING, ECHUNK, SL), jnp.float32),
            pltpu.VMEM_SHARED((N_PAD, SL), jnp.float32),
            pltpu.SemaphoreType.DMA((NRING,)),
            pltpu.SemaphoreType.DMA((NRING,)),
        ],
    )


def _sc_gather_body(t0, t1, t2a, t2b, h2, vcat_hbm,
                    u0, u1, u2a, u2b, uh2,
                    vcat_v, gbuf, sem):
    c = lax.axis_index("c")
    s = lax.axis_index("s")
    vbase = s * VPT
    pltpu.sync_copy(vcat_hbm.at[pl.ds(vbase, VPT)], vcat_v)

    def run(table, out):
        for k in range(VPT // ECHUNK):
            pltpu.async_copy(
                table.at[vcat_v.at[pl.ds(k * ECHUNK, ECHUNK)]],
                gbuf, sem).wait()
            pltpu.sync_copy(
                gbuf, out.at[pl.ds(vbase + k * ECHUNK, ECHUNK)])

    @pl.when(c == 0)
    def _():
        run(t0, u0)
        run(t2a, u2a)
        run(h2, uh2)

    @pl.when(c == 1)
    def _():
        run(t1, u1)
        run(t2b, u2b)


@functools.cache
def _sc_gather():
    st = jax.ShapeDtypeStruct((BB, SL), jnp.float32)
    return pl.kernel(
        _sc_gather_body,
        mesh=plsc.VectorSubcoreMesh(**_SC_MESH),
        compiler_params=pltpu.CompilerParams(use_tc_tiling_on_sc=False),
        out_type=(st, st, st, st, st),
        scratch_types=[
            pltpu.VMEM((VPT,), jnp.int32),
            pltpu.VMEM((ECHUNK, SL), jnp.float32),
            pltpu.SemaphoreType.DMA,
        ],
    )


def _tc_slice_body(x_ref, o0, o1, o2):
    x = x_ref[...]
    o0[...] = x[:, 0 * SL:1 * SL]
    o1[...] = x[:, 1 * SL:2 * SL]
    o2[...] = x[:, 2 * SL:3 * SL]


def _tc_slice(xp):
    blk = 512
    st = jax.ShapeDtypeStruct((N_PAD, SL), jnp.float32)
    bs = pl.BlockSpec((blk, SL), lambda i: (i, 0))
    return pl.pallas_call(
        _tc_slice_body,
        grid=(N_PAD // blk,),
        in_specs=[pl.BlockSpec((blk, D_PAD), lambda i: (i, 0))],
        out_specs=[bs, bs, bs],
        out_shape=[st, st, st],
    )(xp)


def _tc_linear_body(p0, p1, p2a, p2b, x2_ref, w_ref, b_ref, o0, o1, o2):
    sblk = jnp.concatenate(
        [p0[...], p1[...], p2a[...] + p2b[...] - x2_ref[...]], axis=1)
    acc = jnp.dot(sblk, w_ref[...], preferred_element_type=jnp.float32)
    h = _leaky(acc + b_ref[...])
    o0[...] = h[:, 0 * SL:1 * SL]
    o1[...] = h[:, 1 * SL:2 * SL]
    o2[...] = h[:, 2 * SL:3 * SL]


def _tc_linear(p0, p1, p2a, p2b, x2, wt, b):
    n = p0.shape[0]
    blk = 512
    st = jax.ShapeDtypeStruct((n, SL), jnp.float32)
    bs = pl.BlockSpec((blk, SL), lambda i: (i, 0))
    return pl.pallas_call(
        _tc_linear_body,
        grid=(n // blk,),
        in_specs=[bs, bs, bs, bs, bs,
                  pl.BlockSpec((D_PAD, D_PAD), lambda i: (0, 0)),
                  pl.BlockSpec((1, D_PAD), lambda i: (0, 0))],
        out_specs=[bs, bs, bs],
        out_shape=[st, st, st],
    )(p0, p1, p2a, p2b, x2, wt, b)


def _tc_final_body(u0, u1, u2a, u2b, uh2, w2_ref, b2_ref, w3_ref, b3_ref,
                   o_ref):
    u = jnp.concatenate(
        [u0[...], u1[...], u2a[...] + u2b[...] - uh2[...]], axis=1)
    t = jnp.dot(u, w2_ref[...], preferred_element_type=jnp.float32)
    t = _leaky(t + b2_ref[...])
    z = jnp.dot(t, w3_ref[...], preferred_element_type=jnp.float32)
    z = _leaky(z + b3_ref[...])
    nrm = jnp.sqrt(jnp.sum(z * z, axis=1, keepdims=True))
    zn = z / jnp.maximum(nrm, 1e-12)
    o_ref[...] = zn[:, :OUT]


def _tc_final(u0, u1, u2a, u2b, uh2, w2t, b2, w3t, b3):
    blk = 512
    bs = pl.BlockSpec((blk, SL), lambda i: (i, 0))
    return pl.pallas_call(
        _tc_final_body,
        grid=(BB // blk,),
        in_specs=[bs, bs, bs, bs, bs,
                  pl.BlockSpec((D_PAD, D_PAD), lambda i: (0, 0)),
                  pl.BlockSpec((1, D_PAD), lambda i: (0, 0)),
                  pl.BlockSpec((D_PAD, OUT_PAD), lambda i: (0, 0)),
                  pl.BlockSpec((1, OUT_PAD), lambda i: (0, 0))],
        out_specs=pl.BlockSpec((blk, OUT), lambda i: (i, 0)),
        out_shape=jax.ShapeDtypeStruct((BB, OUT), jnp.float32),
    )(u0, u1, u2a, u2b, uh2, w2t, b2, w3t, b3)


def kernel(features, edge_index, v1, v2, W1, b1, W2, b2, W3, b3):
    xp = jnp.pad(features, ((0, N_PAD - N), (0, D_PAD - D)))
    x0, x1, x2 = _tc_slice(xp)
    src = jnp.pad(edge_index[0], (0, E_PAD - E))
    dst = jnp.pad(edge_index[1], (0, E_PAD - E), constant_values=N_PAD - 1)
    vcat = jnp.concatenate([v1, v2])

    w1t = jnp.pad(W1, ((0, D_PAD - D), (0, D_PAD - D))).T
    b1p = jnp.pad(b1, (0, D_PAD - D)).reshape(1, D_PAD)
    w2t = jnp.pad(W2, ((0, D_PAD - D), (0, D_PAD - D))).T
    b2p = jnp.pad(b2, (0, D_PAD - D)).reshape(1, D_PAD)
    w3t = jnp.pad(W3, ((0, OUT_PAD - OUT), (0, D_PAD - D))).T
    b3p = jnp.pad(b3, (0, OUT_PAD - OUT)).reshape(1, OUT_PAD)

    s0, s1, s2a, s2b = _sc_segsum()(x0, x1, x2, src, dst)
    h0, h1, h2 = _tc_linear(s0, s1, s2a, s2b, x2, w1t, b1p)

    t0, t1, t2a, t2b = _sc_segsum()(h0, h1, h2, src, dst)

    u0, u1, u2a, u2b, uh2 = _sc_gather()(t0, t1, t2a, t2b, h2, vcat)
    z = _tc_final(u0, u1, u2a, u2b, uh2, w2t, b2p, w3t, b3p)
    return (z[:B], z[B:])


# speedup vs baseline: 1.1470x; 1.1403x over previous
import functools

import jax
import jax.numpy as jnp
import numpy as np
from jax import lax
from jax.experimental import pallas as pl
from jax.experimental.pallas import tpu as pltpu
from jax.experimental.pallas import tpu_sc as plsc

N = 10000
E = 160000
D = 364
OUT = 150
B = 4096

N_PAD = 10240
D_PAD = 384
HALF = 192
QUAR = 96
E_PAD = 163840
EPT = E_PAD // 16
ECHUNK = 64
NRING = 4
RPT = N_PAD // 16
OUT_PAD = 256
BB = 2 * B
VPT = BB // 16

_SC_MESH = dict(core_axis_name="c", subcore_axis_name="s")


def _leaky(x):
    return jnp.where(x > 0, x, 0.01 * x)


def _sc_segsum_body(xqh, src_hbm, dst_hbm, out_hbm,
                    src_all, dst_all, gidx, didx, gbufh, gbuf, acc,
                    gsem, ssem):
    c = lax.axis_index("c")
    s = lax.axis_index("s")
    iota = lax.iota(jnp.int32, 16)
    ngrp = ECHUNK // 16

    ebase = s * EPT
    pltpu.sync_copy(src_hbm.at[pl.ds(ebase, EPT)], src_all)
    pltpu.sync_copy(dst_hbm.at[pl.ds(ebase, EPT)], dst_all)
    rbase = s * RPT
    nchunks = EPT // ECHUNK

    def convert(t):
        def crow(r, _):
            for g in range(QUAR // 32):
                v = gbufh[t, r, pl.ds(32 * g, 32)]
                lo, hi = plsc.unpack(v, format=plsc.PackFormat.INTERLEAVED)
                gbuf[t, r, pl.ds(32 * g, 16)] = lo
                gbuf[t, r, pl.ds(32 * g + 16, 16)] = hi
            return 0
        lax.fori_loop(0, ECHUNK, crow, 0)

    for phase in range(2):
        q = 2 * phase + c

        for i in range(RPT // ECHUNK):
            b = i % 2
            for j in range(ngrp):
                rows = iota + (rbase + i * ECHUNK + j * 16)
                gidx[b, pl.ds(j * 16, 16)] = rows * 4 + q
            pltpu.async_copy(xqh.at[gidx.at[b]], gbufh.at[b],
                             gsem.at[b]).wait()
            convert(b)
            pltpu.sync_copy(gbuf.at[b],
                            acc.at[pl.ds(rbase + i * ECHUNK, ECHUNK)])
        plsc.subcore_barrier()

        def fill(i, t):
            for j in range(ngrp):
                off = i * ECHUNK + j * 16
                gidx[t, pl.ds(j * 16, 16)] = src_all[pl.ds(off, 16)] * 4 + q
                didx[t, pl.ds(j * 16, 16)] = dst_all[pl.ds(off, 16)]

        def gather(t):
            pltpu.async_copy(xqh.at[gidx.at[t]], gbufh.at[t], gsem.at[t])

        def gather_wait(t):
            pltpu.make_async_copy(
                xqh.at[gidx.at[t]], gbufh.at[t], gsem.at[t]).wait()

        def scatter(t):
            pltpu.async_copy(gbuf.at[t], acc.at[didx.at[t]], ssem.at[t],
                             add=True)

        def scatter_wait(t):
            pltpu.make_async_copy(gbuf.at[t], acc.at[didx.at[t]],
                                  ssem.at[t]).wait()

        fill(0, 0)
        gather(0)

        def group(g, _):
            i0 = NRING * g
            for t in range(NRING):
                i = i0 + t
                nxt = (t + 1) % NRING

                @pl.when(i + 1 < nchunks)
                def _():
                    @pl.when(i + 1 >= NRING)
                    def _():
                        scatter_wait(nxt)
                    fill(i + 1, nxt)
                    gather(nxt)
                gather_wait(t)
                convert(t)
                scatter(t)
            return 0

        lax.fori_loop(0, nchunks // NRING, group, 0)
        for t in range(NRING):
            scatter_wait(t)
        plsc.subcore_barrier()

        for i in range(RPT // ECHUNK):
            b = i % 2
            pltpu.sync_copy(acc.at[pl.ds(rbase + i * ECHUNK, ECHUNK)],
                            gbuf.at[b])
            for j in range(ngrp):
                rows = iota + (rbase + i * ECHUNK + j * 16)
                gidx[b, pl.ds(j * 16, 16)] = rows * 4 + q
            pltpu.async_copy(gbuf.at[b], out_hbm.at[gidx.at[b]],
                             gsem.at[b]).wait()


@functools.cache
def _sc_segsum():
    return pl.kernel(
        _sc_segsum_body,
        mesh=plsc.VectorSubcoreMesh(**_SC_MESH),
        compiler_params=pltpu.CompilerParams(use_tc_tiling_on_sc=False,
                                            needs_layout_passes=False),
        out_type=jax.ShapeDtypeStruct((4 * N_PAD, QUAR), jnp.float32),
        scratch_types=[
            pltpu.VMEM((EPT,), jnp.int32),
            pltpu.VMEM((EPT,), jnp.int32),
            pltpu.VMEM((NRING, ECHUNK), jnp.int32),
            pltpu.VMEM((NRING, ECHUNK), jnp.int32),
            pltpu.VMEM((NRING, ECHUNK, QUAR), jnp.bfloat16),
            pltpu.VMEM((NRING, ECHUNK, QUAR), jnp.float32),
            pltpu.VMEM_SHARED((N_PAD, QUAR), jnp.float32),
            pltpu.SemaphoreType.DMA((NRING,)),
            pltpu.SemaphoreType.DMA((NRING,)),
        ],
    )


def _sc_gather_body(s2r, vcat_hbm, out_hbm, vcat_v, gidx, oidx, gbuf, sem):
    c = lax.axis_index("c")
    s = lax.axis_index("s")
    iota = lax.iota(jnp.int32, 16)
    vbase = s * VPT
    pltpu.sync_copy(vcat_hbm.at[pl.ds(vbase, VPT)], vcat_v)
    for k in range(VPT // ECHUNK):
        for j in range(ECHUNK // 16):
            off = k * ECHUNK + j * 16
            vv = vcat_v[pl.ds(off, 16)]
            gidx[pl.ds(j * 16, 16)] = vv * 2 + c
            oidx[pl.ds(j * 16, 16)] = (iota + vbase + off) * 2 + c
        pltpu.async_copy(s2r.at[gidx], gbuf, sem).wait()
        pltpu.async_copy(gbuf, out_hbm.at[oidx], sem).wait()


@functools.cache
def _sc_gather():
    return pl.kernel(
        _sc_gather_body,
        mesh=plsc.VectorSubcoreMesh(**_SC_MESH),
        compiler_params=pltpu.CompilerParams(use_tc_tiling_on_sc=False),
        out_type=jax.ShapeDtypeStruct((2 * BB, HALF), jnp.float32),
        scratch_types=[
            pltpu.VMEM((VPT,), jnp.int32),
            pltpu.VMEM((ECHUNK,), jnp.int32),
            pltpu.VMEM((ECHUNK,), jnp.int32),
            pltpu.VMEM((ECHUNK, HALF), jnp.float32),
            pltpu.SemaphoreType.DMA,
        ],
    )


def _tc_linear_body(s_ref, w_ref, b_ref, ob_ref):
    acc = jnp.dot(s_ref[...], w_ref[...], preferred_element_type=jnp.float32)
    ob_ref[...] = _leaky(acc + b_ref[...]).astype(jnp.bfloat16)


def _tc_linear(sarr, wt, b):
    n = sarr.shape[0]
    blk = 512
    return pl.pallas_call(
        _tc_linear_body,
        grid=(n // blk,),
        in_specs=[
            pl.BlockSpec((blk, D_PAD), lambda i: (i, 0)),
            pl.BlockSpec((D_PAD, D_PAD), lambda i: (0, 0)),
            pl.BlockSpec((1, D_PAD), lambda i: (0, 0)),
        ],
        out_specs=pl.BlockSpec((blk, D_PAD), lambda i: (i, 0)),
        out_shape=jax.ShapeDtypeStruct((n, D_PAD), jnp.bfloat16),
    )(sarr, wt, b)


def _tc_cast_body(x_ref, o_ref):
    o_ref[...] = x_ref[...].astype(jnp.bfloat16)


def _tc_cast(xp):
    blk = 512
    return pl.pallas_call(
        _tc_cast_body,
        grid=(N_PAD // blk,),
        in_specs=[pl.BlockSpec((blk, D_PAD), lambda i: (i, 0))],
        out_specs=pl.BlockSpec((blk, D_PAD), lambda i: (i, 0)),
        out_shape=jax.ShapeDtypeStruct((N_PAD, D_PAD), jnp.bfloat16),
    )(xp)


def _tc_final_body(u_ref, w2_ref, b2_ref, w3_ref, b3_ref, o_ref):
    t = jnp.dot(u_ref[...], w2_ref[...], preferred_element_type=jnp.float32)
    t = _leaky(t + b2_ref[...])
    z = jnp.dot(t, w3_ref[...], preferred_element_type=jnp.float32)
    z = _leaky(z + b3_ref[...])
    n = jnp.sqrt(jnp.sum(z * z, axis=1, keepdims=True))
    o_ref[...] = z / jnp.maximum(n, 1e-12)


def _tc_final(u, w2t, b2, w3t, b3):
    blk = 512
    return pl.pallas_call(
        _tc_final_body,
        grid=(BB // blk,),
        in_specs=[
            pl.BlockSpec((blk, D_PAD), lambda i: (i, 0)),
            pl.BlockSpec((D_PAD, D_PAD), lambda i: (0, 0)),
            pl.BlockSpec((1, D_PAD), lambda i: (0, 0)),
            pl.BlockSpec((D_PAD, OUT_PAD), lambda i: (0, 0)),
            pl.BlockSpec((1, OUT_PAD), lambda i: (0, 0)),
        ],
        out_specs=pl.BlockSpec((blk, OUT_PAD), lambda i: (i, 0)),
        out_shape=jax.ShapeDtypeStruct((BB, OUT_PAD), jnp.float32),
    )(u, w2t, b2, w3t, b3)


_PI = np.arange(D_PAD, dtype=np.int32).reshape(-1, 16, 2).transpose(
    0, 2, 1).reshape(-1)


def kernel(features, edge_index, v1, v2, W1, b1, W2, b2, W3, b3):
    xp = jnp.pad(features, ((0, N_PAD - N), (0, D_PAD - D)))
    src = jnp.pad(edge_index[0], (0, E_PAD - E))
    dst = jnp.pad(edge_index[1], (0, E_PAD - E), constant_values=N_PAD - 1)
    vcat = jnp.concatenate([v1, v2])

    w1t = jnp.pad(W1, ((0, D_PAD - D), (0, D_PAD - D))).T[_PI]
    b1p = jnp.pad(b1, (0, D_PAD - D)).reshape(1, D_PAD)
    w2t = jnp.pad(W2, ((0, D_PAD - D), (0, D_PAD - D))).T[_PI]
    b2p = jnp.pad(b2, (0, D_PAD - D)).reshape(1, D_PAD)
    w3t = jnp.pad(W3, ((0, OUT_PAD - OUT), (0, D_PAD - D))).T
    b3p = jnp.pad(b3, (0, OUT_PAD - OUT)).reshape(1, OUT_PAD)

    xb = _tc_cast(xp)
    s1q = _sc_segsum()(xb.reshape(4 * N_PAD, QUAR), src, dst)
    hb = _tc_linear(s1q.reshape(N_PAD, D_PAD), w1t, b1p)
    s2q = _sc_segsum()(hb.reshape(4 * N_PAD, QUAR), src, dst)
    ur = _sc_gather()(s2q.reshape(2 * N_PAD, HALF), vcat)
    z = _tc_final(ur.reshape(BB, D_PAD), w2t, b2p, w3t, b3p)
    return (z[:B, :OUT], z[B:, :OUT])
